# Initial kernel scaffold; baseline (speedup 1.0000x reference)
import jax
import jax.numpy as jnp
from jax.experimental import pallas as pl


def kernel(x, edge_index, edge_attr, batch, lin1_w, lin1_b, W1a, b1a, W1b, b1b, lin2_w, lin2_b, W2a, b2a, W2b, b2b, fc_w, fc_b, hS_w, hS_b, hP_w, hP_b, hN_w, hN_b):
    raise NotImplementedError("write your pallas kernel here")


# speedup vs baseline: 2.8006x; 2.8006x over previous
import functools

import jax
import jax.numpy as jnp
from jax import lax
from jax.experimental import pallas as pl
from jax.experimental.pallas import tpu as pltpu
from jax.experimental.pallas import tpu_sc as plsc

_NC = 2
_NS = 16
_LANES = 16


def _edge_linear(ea, w1, b1, w2, b2):
    E, DE = ea.shape
    H = w1.shape[1]
    BE = 2000
    grid = (E // BE,)

    def body(ea_ref, w1_ref, b1_ref, w2_ref, b2_ref, o1_ref, o2_ref):
        a = ea_ref[...]
        o1_ref[...] = jnp.dot(a, w1_ref[...], preferred_element_type=jnp.float32) + b1_ref[...]
        o2_ref[...] = jnp.dot(a, w2_ref[...], preferred_element_type=jnp.float32) + b2_ref[...]

    return pl.pallas_call(
        body,
        grid=grid,
        in_specs=[
            pl.BlockSpec((BE, DE), lambda i: (i, 0)),
            pl.BlockSpec((DE, H), lambda i: (0, 0)),
            pl.BlockSpec((1, H), lambda i: (0, 0)),
            pl.BlockSpec((DE, H), lambda i: (0, 0)),
            pl.BlockSpec((1, H), lambda i: (0, 0)),
        ],
        out_specs=[
            pl.BlockSpec((BE, H), lambda i: (i, 0)),
            pl.BlockSpec((BE, H), lambda i: (i, 0)),
        ],
        out_shape=[jax.ShapeDtypeStruct((E, H), jnp.float32)] * 2,
    )(ea, w1, b1[None, :], w2, b2[None, :])


def _sc_message_aggr(x, e, src, dst, zrows):
    N, H = x.shape
    E = src.shape[0]
    C = 128
    NCH = E // C
    W = _NC * _NS
    ITERS = (NCH + W - 1) // W
    RPT = (N // _NS) // 8 * 8
    REM = N - RPT * _NS
    mesh = plsc.VectorSubcoreMesh(core_axis_name="c", subcore_axis_name="s")

    @functools.partial(
        pl.kernel,
        out_type=jax.ShapeDtypeStruct((_NC * N, H), jnp.float32),
        mesh=mesh,
        scratch_types=[
            pltpu.VMEM((C,), jnp.int32),
            pltpu.VMEM((C,), jnp.int32),
            pltpu.VMEM((C, H), jnp.float32),
            pltpu.VMEM((C, H), jnp.float32),
            pltpu.VMEM_SHARED((N, H), jnp.float32),
            pltpu.SemaphoreType.DMA,
        ],
    )
    def k(x_hbm, e_hbm, src_hbm, dst_hbm, z_hbm, out_hbm, srcv, dstv, xr, er, acc, sem):
        cid = lax.axis_index("c")
        sid = lax.axis_index("s")
        wid = sid * _NC + cid
        pltpu.sync_copy(z_hbm.at[pl.ds(0, RPT)], acc.at[pl.ds(sid * RPT, RPT)])
        if REM:
            @pl.when(sid == _NS - 1)
            def _():
                pltpu.sync_copy(z_hbm.at[pl.ds(0, REM)],
                                acc.at[pl.ds(_NS * RPT, REM)])
        plsc.subcore_barrier()

        def chunk(i, carry):
            ch = i * W + wid

            @pl.when(ch < NCH)
            def _():
                base = ch * C
                pltpu.sync_copy(src_hbm.at[pl.ds(base, C)], srcv)
                pltpu.sync_copy(dst_hbm.at[pl.ds(base, C)], dstv)
                pltpu.sync_copy(e_hbm.at[pl.ds(base, C)], er)
                pltpu.async_copy(x_hbm.at[srcv], xr, sem).wait()

                def row(r, c2):
                    for j in range(H // _LANES):
                        sl = pl.ds(j * _LANES, _LANES)
                        xr[r, sl] = jnp.maximum(xr[r, sl] + er[r, sl], 0.0)
                    return c2

                lax.fori_loop(0, C, row, 0)
                pltpu.sync_copy(xr, acc.at[dstv], add=True)

            return carry

        lax.fori_loop(0, ITERS, chunk, 0)
        plsc.subcore_barrier()
        pltpu.sync_copy(acc.at[pl.ds(sid * RPT, RPT)],
                        out_hbm.at[pl.ds(cid * N + sid * RPT, RPT)])
        if REM:
            @pl.when(sid == _NS - 1)
            def _():
                pltpu.sync_copy(acc.at[pl.ds(_NS * RPT, REM)],
                                out_hbm.at[pl.ds(cid * N + _NS * RPT, REM)])

    return k(x, e, src, dst, zrows)


def _node_mlp(x, agg, Wa, ba, Wb, bb):
    N, H = x.shape
    BN = 1000
    nb = N // BN

    def body(x_ref, a0_ref, a1_ref, wa_ref, ba_ref, wb_ref, bb_ref, o_ref):
        h = x_ref[...] + a0_ref[...] + a1_ref[...]
        h = jnp.maximum(jnp.dot(h, wa_ref[...], preferred_element_type=jnp.float32) + ba_ref[...], 0.0)
        h = jnp.dot(h, wb_ref[...], preferred_element_type=jnp.float32) + bb_ref[...]
        o_ref[...] = jnp.maximum(h, 0.0)

    return pl.pallas_call(
        body,
        grid=(nb,),
        in_specs=[
            pl.BlockSpec((BN, H), lambda i: (i, 0)),
            pl.BlockSpec((BN, H), lambda i: (i, 0)),
            pl.BlockSpec((BN, H), lambda i: (i + nb, 0)),
            pl.BlockSpec((H, H), lambda i: (0, 0)),
            pl.BlockSpec((1, H), lambda i: (0, 0)),
            pl.BlockSpec((H, H), lambda i: (0, 0)),
            pl.BlockSpec((1, H), lambda i: (0, 0)),
        ],
        out_specs=pl.BlockSpec((BN, H), lambda i: (i, 0)),
        out_shape=jax.ShapeDtypeStruct((N, H), jnp.float32),
    )(x, agg, agg, Wa, ba[None, :], Wb, bb[None, :])


def _pool_heads(h, batch2, fc_w, fc_b, hw, hb, G):
    N, H = h.shape
    S = fc_w.shape[1]
    BN = 1000
    nb = N // BN

    def body(h_ref, b_ref, fcw_ref, fcb_ref, hw_ref, hb_ref, o_ref, sums, counts):
        i = pl.program_id(0)

        @pl.when(i == 0)
        def _():
            sums[...] = jnp.zeros_like(sums)
            counts[...] = jnp.zeros_like(counts)

        oh = (b_ref[...] == lax.broadcasted_iota(jnp.int32, (1, G), 1)).astype(jnp.float32)
        hblk = h_ref[...]
        sums[...] += lax.dot_general(oh, hblk, (((0,), (0,)), ((), ())),
                                     preferred_element_type=jnp.float32)
        counts[...] += lax.dot_general(oh, jnp.ones((BN, 1), jnp.float32),
                                       (((0,), (0,)), ((), ())),
                                       preferred_element_type=jnp.float32)

        @pl.when(i == nb - 1)
        def _():
            pooled = sums[...] / jnp.maximum(counts[...], 1.0)
            sh = jnp.maximum(jnp.dot(pooled, fcw_ref[...], preferred_element_type=jnp.float32)
                             + fcb_ref[...], 0.0)
            o_ref[...] = jnp.dot(sh, hw_ref[...], preferred_element_type=jnp.float32) + hb_ref[...]

    return pl.pallas_call(
        body,
        grid=(nb,),
        in_specs=[
            pl.BlockSpec((BN, H), lambda i: (i, 0)),
            pl.BlockSpec((BN, 1), lambda i: (i, 0)),
            pl.BlockSpec((H, S), lambda i: (0, 0)),
            pl.BlockSpec((1, S), lambda i: (0, 0)),
            pl.BlockSpec((S, 128), lambda i: (0, 0)),
            pl.BlockSpec((1, 128), lambda i: (0, 0)),
        ],
        out_specs=pl.BlockSpec((G, 128), lambda i: (0, 0)),
        out_shape=jax.ShapeDtypeStruct((G, 128), jnp.float32),
        scratch_shapes=[
            pltpu.VMEM((G, H), jnp.float32),
            pltpu.VMEM((G, 1), jnp.float32),
        ],
    )(h, batch2, fc_w, fc_b[None, :], hw, hb)


def kernel(x, edge_index, edge_attr, batch,
           lin1_w, lin1_b, W1a, b1a, W1b, b1b,
           lin2_w, lin2_b, W2a, b2a, W2b, b2b,
           fc_w, fc_b, hS_w, hS_b, hP_w, hP_b, hN_w, hN_b):
    src = edge_index[0]
    dst = edge_index[1]
    N, D = x.shape
    H = W1a.shape[1]
    G = 64

    e1, e2 = _edge_linear(edge_attr, lin1_w, lin1_b, lin2_w, lin2_b)
    z = jnp.zeros(((N // _NS) // 8 * 8, H), jnp.float32)

    agg1 = _sc_message_aggr(x, e1, src, dst, z)
    h1 = _node_mlp(x, agg1, W1a, b1a, W1b, b1b)

    agg2 = _sc_message_aggr(h1, e2, src, dst, z)
    h2 = _node_mlp(h1, agg2, W2a, b2a, W2b, b2b)

    hw = jnp.pad(jnp.concatenate([hS_w, hP_w, hN_w], axis=1), ((0, 0), (0, 125)))
    hb = jnp.pad(jnp.concatenate([hS_b, hP_b, hN_b], axis=0), (0, 125))[None, :]
    out = _pool_heads(h2, batch[:, None], fc_w, fc_b, hw, hb, G)
    return out[:, 0], out[:, 1], out[:, 2]
